# baseline (speedup 1.0000x reference)
import functools

import jax
import jax.numpy as jnp
from jax import lax
from jax.experimental import pallas as pl
from jax.experimental.pallas import tpu as pltpu
from jax.experimental.pallas import tpu_sc as plsc

_NC = 2
_NS = 16
_NW = _NC * _NS
_L = 16


def _sc_means_body(src_hbm, nbr_hbm, means_hbm, nbr_v, src_v, out_v, *, k, b, d, n_blk):
    nvec = d // _L
    inv = 1.0 / (k + 1)
    cid = lax.axis_index("c")
    sid = lax.axis_index("s")
    wid = sid * _NC + cid
    my_blk = (n_blk - wid + _NW - 1) // _NW

    def jbody(j, carry):
        blk = wid + _NW * j
        pltpu.sync_copy(nbr_hbm.at[pl.ds(blk * b * k, b * k)], nbr_v)
        pltpu.sync_copy(src_hbm.at[pl.ds(blk * b, b)], src_v)
        for seg in range(b):
            accs = tuple(src_v[seg, pl.ds(c * _L, _L)] for c in range(nvec))

            def rbody(r, accs):
                row = seg * k + r
                return tuple(
                    a + nbr_v[row, pl.ds(c * _L, _L)] for c, a in enumerate(accs)
                )

            accs = lax.fori_loop(0, k, rbody, accs)
            for c in range(nvec):
                out_v[seg, pl.ds(c * _L, _L)] = accs[c] * inv
        pltpu.sync_copy(out_v, means_hbm.at[pl.ds(blk * b, b)])
        return carry

    lax.fori_loop(0, my_blk, jbody, 0)


def _matmul_kernel(means_ref, w_ref, out_ref):
    out_ref[...] = jax.nn.relu(
        jnp.dot(means_ref[...], w_ref[...], preferred_element_type=jnp.float32)
    )


def kernel(src_vectors, neighbor_vectors, W):
    n_src, d = src_vectors.shape
    n_nbr = neighbor_vectors.shape[0]
    out_dim = W.shape[1]
    k = n_nbr // n_src

    b = 8
    n_blk = n_src // b
    mesh = plsc.VectorSubcoreMesh(core_axis_name="c", subcore_axis_name="s")

    means = pl.kernel(
        functools.partial(_sc_means_body, k=k, b=b, d=d, n_blk=n_blk),
        out_type=jax.ShapeDtypeStruct((n_src, d), jnp.float32),
        mesh=mesh,
        scratch_types=[
            pltpu.VMEM((b * k, d), jnp.float32),
            pltpu.VMEM((b, d), jnp.float32),
            pltpu.VMEM((b, d), jnp.float32),
        ],
    )(src_vectors, neighbor_vectors)

    S = 1000
    return pl.pallas_call(
        _matmul_kernel,
        grid=(n_src // S,),
        in_specs=[
            pl.BlockSpec((S, d), lambda i: (i, 0)),
            pl.BlockSpec((d, out_dim), lambda i: (0, 0)),
        ],
        out_specs=pl.BlockSpec((S, out_dim), lambda i: (i, 0)),
        out_shape=jax.ShapeDtypeStruct((n_src, out_dim), jnp.float32),
    )(means, W)


# speedup vs baseline: 1.6071x; 1.6071x over previous
import functools

import jax
import jax.numpy as jnp
from jax import lax
from jax.experimental import pallas as pl
from jax.experimental.pallas import tpu as pltpu
from jax.experimental.pallas import tpu_sc as plsc

_NC = 2
_NS = 16
_NW = _NC * _NS
_L = 16


def _sc_means_body(src_hbm, nbr_hbm, means_hbm,
                   nbr_a, nbr_b, src_a, src_b, out_a, out_b,
                   sem_in_a, sem_in_b, sem_out_a, sem_out_b,
                   *, k, b, d, n_blk):
    nvec = d // _L
    inv = 1.0 / (k + 1)
    wid = lax.axis_index("s") * _NC + lax.axis_index("c")
    my_blk = (n_blk - wid + _NW - 1) // _NW

    def start_in(j, nbr_v, src_v, sem):
        blk = wid + _NW * j
        pltpu.async_copy(nbr_hbm.at[pl.ds(blk * b * k, b * k)], nbr_v, sem)
        pltpu.async_copy(src_hbm.at[pl.ds(blk * b, b)], src_v, sem)

    def wait_in(nbr_v, src_v, sem):
        pltpu.make_async_copy(nbr_hbm.at[pl.ds(0, b * k)], nbr_v, sem).wait()
        pltpu.make_async_copy(src_hbm.at[pl.ds(0, b)], src_v, sem).wait()

    def start_out(j, out_v, sem):
        blk = wid + _NW * j
        pltpu.async_copy(out_v, means_hbm.at[pl.ds(blk * b, b)], sem)

    def wait_out(out_v, sem):
        pltpu.make_async_copy(out_v, means_hbm.at[pl.ds(0, b)], sem).wait()

    def compute(nbr_v, src_v, out_v):
        for seg in range(b):
            accs = tuple(src_v[seg, pl.ds(c * _L, _L)] for c in range(nvec))
            if k % 4 == 0:
                def rbody(r, accs):
                    row = seg * k + r * 4
                    for u in range(4):
                        accs = tuple(
                            a + nbr_v[row + u, pl.ds(c * _L, _L)]
                            for c, a in enumerate(accs)
                        )
                    return accs
                accs = lax.fori_loop(0, k // 4, rbody, accs)
            else:
                def rbody(r, accs):
                    row = seg * k + r
                    return tuple(
                        a + nbr_v[row, pl.ds(c * _L, _L)]
                        for c, a in enumerate(accs)
                    )
                accs = lax.fori_loop(0, k, rbody, accs)
            for c in range(nvec):
                out_v[seg, pl.ds(c * _L, _L)] = accs[c] * inv

    @pl.when(my_blk > 0)
    def _():
        start_in(0, nbr_a, src_a, sem_in_a)

    def pbody(p, carry):
        j0 = 2 * p
        j1 = j0 + 1

        wait_in(nbr_a, src_a, sem_in_a)

        @pl.when(j1 < my_blk)
        def _():
            start_in(j1, nbr_b, src_b, sem_in_b)

        @pl.when(j0 >= 2)
        def _():
            wait_out(out_a, sem_out_a)

        compute(nbr_a, src_a, out_a)
        start_out(j0, out_a, sem_out_a)

        @pl.when(j1 < my_blk)
        def _():
            wait_in(nbr_b, src_b, sem_in_b)

            @pl.when(j1 + 1 < my_blk)
            def _():
                start_in(j1 + 1, nbr_a, src_a, sem_in_a)

            @pl.when(j1 >= 2)
            def _():
                wait_out(out_b, sem_out_b)

            compute(nbr_b, src_b, out_b)
            start_out(j1, out_b, sem_out_b)

        return carry

    npair = (my_blk + 1) // 2
    lax.fori_loop(0, npair, pbody, 0)

    @pl.when(my_blk >= 1)
    def _():
        wait_out(out_a, sem_out_a)

    @pl.when(my_blk >= 2)
    def _():
        wait_out(out_b, sem_out_b)


def _matmul_kernel(means_ref, w_ref, out_ref):
    out_ref[...] = jax.nn.relu(
        jnp.dot(means_ref[...], w_ref[...], preferred_element_type=jnp.float32)
    )


def kernel(src_vectors, neighbor_vectors, W):
    n_src, d = src_vectors.shape
    n_nbr = neighbor_vectors.shape[0]
    out_dim = W.shape[1]
    k = n_nbr // n_src

    b = 8
    n_blk = n_src // b
    mesh = plsc.VectorSubcoreMesh(core_axis_name="c", subcore_axis_name="s")

    means = pl.kernel(
        functools.partial(_sc_means_body, k=k, b=b, d=d, n_blk=n_blk),
        out_type=jax.ShapeDtypeStruct((n_src, d), jnp.float32),
        mesh=mesh,
        scratch_types=[
            pltpu.VMEM((b * k, d), jnp.float32),
            pltpu.VMEM((b * k, d), jnp.float32),
            pltpu.VMEM((b, d), jnp.float32),
            pltpu.VMEM((b, d), jnp.float32),
            pltpu.VMEM((b, d), jnp.float32),
            pltpu.VMEM((b, d), jnp.float32),
            pltpu.SemaphoreType.DMA,
            pltpu.SemaphoreType.DMA,
            pltpu.SemaphoreType.DMA,
            pltpu.SemaphoreType.DMA,
        ],
    )(src_vectors, neighbor_vectors)

    S = 1000
    return pl.pallas_call(
        _matmul_kernel,
        grid=(n_src // S,),
        in_specs=[
            pl.BlockSpec((S, d), lambda i: (i, 0)),
            pl.BlockSpec((d, out_dim), lambda i: (0, 0)),
        ],
        out_specs=pl.BlockSpec((S, out_dim), lambda i: (i, 0)),
        out_shape=jax.ShapeDtypeStruct((n_src, out_dim), jnp.float32),
    )(means, W)


# speedup vs baseline: 2.3089x; 1.4367x over previous
import functools

import jax
import jax.numpy as jnp
from jax import lax
from jax.experimental import pallas as pl
from jax.experimental.pallas import tpu as pltpu
from jax.experimental.pallas import tpu_sc as plsc

_NC = 2
_NS = 16
_NW = _NC * _NS
_L = 16


def _sc_means_body(src_hbm, nbr_hbm, means_hbm,
                   nbr_a, nbr_b, src_a, src_b, out_a, out_b,
                   sem_in_a, sem_in_b, sem_out_a, sem_out_b,
                   *, k, b, d, blk0, n_blk):
    nvec = d // _L
    inv = 1.0 / (k + 1)
    wid = lax.axis_index("s") * _NC + lax.axis_index("c")
    my_blk = (n_blk - blk0 - wid + _NW - 1) // _NW

    def start_in(j, nbr_v, src_v, sem):
        blk = blk0 + wid + _NW * j
        pltpu.async_copy(nbr_hbm.at[pl.ds(blk * b * k, b * k)], nbr_v, sem)
        pltpu.async_copy(src_hbm.at[pl.ds(blk * b, b)], src_v, sem)

    def wait_in(nbr_v, src_v, sem):
        pltpu.make_async_copy(nbr_hbm.at[pl.ds(0, b * k)], nbr_v, sem).wait()
        pltpu.make_async_copy(src_hbm.at[pl.ds(0, b)], src_v, sem).wait()

    def start_out(j, out_v, sem):
        blk = wid + _NW * j
        pltpu.async_copy(out_v, means_hbm.at[pl.ds(blk * b, b)], sem)

    def wait_out(out_v, sem):
        pltpu.make_async_copy(out_v, means_hbm.at[pl.ds(0, b)], sem).wait()

    def compute(nbr_v, src_v, out_v):
        for seg in range(b):
            accs = tuple(src_v[seg, pl.ds(c * _L, _L)] for c in range(nvec))
            if k % 4 == 0:
                def rbody(r, accs):
                    row = seg * k + r * 4
                    for u in range(4):
                        accs = tuple(
                            a + nbr_v[row + u, pl.ds(c * _L, _L)]
                            for c, a in enumerate(accs)
                        )
                    return accs
                accs = lax.fori_loop(0, k // 4, rbody, accs)
            else:
                def rbody(r, accs):
                    row = seg * k + r
                    return tuple(
                        a + nbr_v[row, pl.ds(c * _L, _L)]
                        for c, a in enumerate(accs)
                    )
                accs = lax.fori_loop(0, k, rbody, accs)
            for c in range(nvec):
                out_v[seg, pl.ds(c * _L, _L)] = accs[c] * inv

    @pl.when(my_blk > 0)
    def _():
        start_in(0, nbr_a, src_a, sem_in_a)

    def pbody(p, carry):
        j0 = 2 * p
        j1 = j0 + 1

        wait_in(nbr_a, src_a, sem_in_a)

        @pl.when(j1 < my_blk)
        def _():
            start_in(j1, nbr_b, src_b, sem_in_b)

        @pl.when(j0 >= 2)
        def _():
            wait_out(out_a, sem_out_a)

        compute(nbr_a, src_a, out_a)
        start_out(j0, out_a, sem_out_a)

        @pl.when(j1 < my_blk)
        def _():
            wait_in(nbr_b, src_b, sem_in_b)

            @pl.when(j1 + 1 < my_blk)
            def _():
                start_in(j1 + 1, nbr_a, src_a, sem_in_a)

            @pl.when(j1 >= 2)
            def _():
                wait_out(out_b, sem_out_b)

            compute(nbr_b, src_b, out_b)
            start_out(j1, out_b, sem_out_b)

        return carry

    npair = (my_blk + 1) // 2
    lax.fori_loop(0, npair, pbody, 0)

    @pl.when(my_blk >= 1)
    def _():
        wait_out(out_a, sem_out_a)

    @pl.when(my_blk >= 2)
    def _():
        wait_out(out_b, sem_out_b)


def _tc_fused_kernel(src_ref, nbr_ref, w_ref, out_ref, *, k):
    s = src_ref.shape[0]
    d = src_ref.shape[1]
    nsum = jnp.reshape(nbr_ref[...], (s, k, d)).sum(axis=1)
    mean = (nsum + src_ref[...]) * (1.0 / (k + 1))
    out_ref[...] = jax.nn.relu(
        jnp.dot(mean, w_ref[...], preferred_element_type=jnp.float32)
    )


def _matmul_kernel(means_ref, w_ref, out_ref):
    out_ref[...] = jax.nn.relu(
        jnp.dot(means_ref[...], w_ref[...], preferred_element_type=jnp.float32)
    )


def kernel(src_vectors, neighbor_vectors, W):
    n_src, d = src_vectors.shape
    n_nbr = neighbor_vectors.shape[0]
    out_dim = W.shape[1]
    k = n_nbr // n_src

    S = 400
    b = 8
    n_tc = (n_src * 68 // 100) // S * S
    if n_src % S != 0 or (n_src - n_tc) % b != 0:
        S = n_src
        n_tc = n_src
    n_sc = n_src - n_tc

    mesh = plsc.VectorSubcoreMesh(core_axis_name="c", subcore_axis_name="s")

    sc_means = pl.kernel(
        functools.partial(
            _sc_means_body, k=k, b=b, d=d, blk0=n_tc // b, n_blk=n_src // b
        ),
        out_type=jax.ShapeDtypeStruct((max(n_sc, b), d), jnp.float32),
        mesh=mesh,
        scratch_types=[
            pltpu.VMEM((b * k, d), jnp.float32),
            pltpu.VMEM((b * k, d), jnp.float32),
            pltpu.VMEM((b, d), jnp.float32),
            pltpu.VMEM((b, d), jnp.float32),
            pltpu.VMEM((b, d), jnp.float32),
            pltpu.VMEM((b, d), jnp.float32),
            pltpu.SemaphoreType.DMA,
            pltpu.SemaphoreType.DMA,
            pltpu.SemaphoreType.DMA,
            pltpu.SemaphoreType.DMA,
        ],
    )(src_vectors, neighbor_vectors)

    tc_out = pl.pallas_call(
        functools.partial(_tc_fused_kernel, k=k),
        grid=(n_tc // S,),
        in_specs=[
            pl.BlockSpec((S, d), lambda i: (i, 0)),
            pl.BlockSpec((S * k, d), lambda i: (i, 0)),
            pl.BlockSpec((d, out_dim), lambda i: (0, 0)),
        ],
        out_specs=pl.BlockSpec((S, out_dim), lambda i: (i, 0)),
        out_shape=jax.ShapeDtypeStruct((n_tc, out_dim), jnp.float32),
    )(src_vectors, neighbor_vectors, W)

    if n_sc == 0:
        return tc_out

    S2 = 800
    sc_out = pl.pallas_call(
        _matmul_kernel,
        grid=(n_sc // S2 if n_sc % S2 == 0 else 1,),
        in_specs=[
            pl.BlockSpec(
                (S2 if n_sc % S2 == 0 else n_sc, d), lambda i: (i, 0)
            ),
            pl.BlockSpec((d, out_dim), lambda i: (0, 0)),
        ],
        out_specs=pl.BlockSpec(
            (S2 if n_sc % S2 == 0 else n_sc, out_dim), lambda i: (i, 0)
        ),
        out_shape=jax.ShapeDtypeStruct((n_sc, out_dim), jnp.float32),
    )(sc_means[:n_sc], W)

    return jnp.concatenate([tc_out, sc_out], axis=0)


# speedup vs baseline: 3.4083x; 1.4761x over previous
import functools

import jax
import jax.numpy as jnp
from jax.experimental import pallas as pl
from jax.experimental.pallas import tpu as pltpu


def _agg_kernel(src_ref, nbr_ref, w_ref, out_ref, *, k):
    s = src_ref.shape[0]
    d = src_ref.shape[1]
    nbr = nbr_ref[...]
    nsum = jnp.reshape(nbr, (s, k, d)).sum(axis=1)
    mean = (nsum + src_ref[...]) * (1.0 / (k + 1))
    out_ref[...] = jax.nn.relu(
        jnp.dot(mean, w_ref[...], preferred_element_type=jnp.float32)
    )


def kernel(src_vectors, neighbor_vectors, W):
    n_src, d = src_vectors.shape
    n_nbr = neighbor_vectors.shape[0]
    out_dim = W.shape[1]
    k = n_nbr // n_src

    S = 400
    grid = (n_src // S,)

    return pl.pallas_call(
        functools.partial(_agg_kernel, k=k),
        grid=grid,
        in_specs=[
            pl.BlockSpec((S, d), lambda i: (i, 0)),
            pl.BlockSpec((S * k, d), lambda i: (i, 0)),
            pl.BlockSpec((d, out_dim), lambda i: (0, 0)),
        ],
        out_specs=pl.BlockSpec((S, out_dim), lambda i: (i, 0)),
        out_shape=jax.ShapeDtypeStruct((n_src, out_dim), jnp.float32),
        compiler_params=pltpu.CompilerParams(
            dimension_semantics=("parallel",),
        ),
    )(src_vectors, neighbor_vectors, W)
